# baseline (speedup 1.0000x reference)
import jax
import jax.numpy as jnp
from jax.experimental import pallas as pl
from jax.experimental.pallas import tpu as pltpu

_BLOCK_C = 3


def _fused_kernel(emb_ref, W1_ref, b1_ref, W2_ref, b2_ref, g_ref, out_ref, y3_ref):
    C = y3_ref.shape[0]
    i = pl.program_id(0)

    @pl.when(i == 0)
    def _compute_tile():
        x = emb_ref[0:C, :]
        h = jnp.dot(x, W1_ref[...], preferred_element_type=jnp.float32) + b1_ref[...]
        h = jax.nn.gelu(h)
        h = jnp.dot(h, W2_ref[...], preferred_element_type=jnp.float32) + b2_ref[...]
        x = x + h
        ms = jnp.mean(x * x, axis=-1, keepdims=True)
        y = x * jax.lax.rsqrt(ms + 1e-6) * g_ref[...]
        y3_ref[...] = y[:, :, None]

    out_ref[...] = jnp.broadcast_to(
        y3_ref[pl.ds(i * _BLOCK_C, _BLOCK_C)], out_ref.shape
    )


@jax.jit
def kernel(pos, sensor_type, emb, W1, b1, W2, b2, g):
    B, C = pos.shape[0], pos.shape[1]
    D = emb.shape[1]

    x_cdb = pl.pallas_call(
        _fused_kernel,
        grid=(C // _BLOCK_C,),
        in_specs=[
            pl.BlockSpec(emb.shape, lambda i: (0, 0)),
            pl.BlockSpec(W1.shape, lambda i: (0, 0)),
            pl.BlockSpec((1, b1.shape[0]), lambda i: (0, 0)),
            pl.BlockSpec(W2.shape, lambda i: (0, 0)),
            pl.BlockSpec((1, b2.shape[0]), lambda i: (0, 0)),
            pl.BlockSpec((1, g.shape[0]), lambda i: (0, 0)),
        ],
        out_specs=pl.BlockSpec((_BLOCK_C, D, B), lambda i: (i, 0, 0)),
        out_shape=jax.ShapeDtypeStruct((C, D, B), jnp.float32),
        scratch_shapes=[pltpu.VMEM((C, D, 1), jnp.float32)],
    )(emb, W1, b1.reshape(1, -1), W2, b2.reshape(1, -1), g.reshape(1, -1))

    return jnp.transpose(x_cdb, (2, 0, 1))


# speedup vs baseline: 1.0361x; 1.0361x over previous
import jax
import jax.numpy as jnp
from jax.experimental import pallas as pl
from jax.experimental.pallas import tpu as pltpu

_BLOCK_C = 6


def _fused_kernel(emb_ref, W1_ref, b1_ref, W2_ref, b2_ref, g_ref, out_ref, y3_ref):
    C = y3_ref.shape[0]
    i = pl.program_id(0)

    @pl.when(i == 0)
    def _compute_tile():
        x = emb_ref[0:C, :]
        h = jnp.dot(x, W1_ref[...], preferred_element_type=jnp.float32) + b1_ref[...]
        h = jax.nn.gelu(h)
        h = jnp.dot(h, W2_ref[...], preferred_element_type=jnp.float32) + b2_ref[...]
        x = x + h
        ms = jnp.mean(x * x, axis=-1, keepdims=True)
        y = x * jax.lax.rsqrt(ms + 1e-6) * g_ref[...]
        y3_ref[...] = y[:, :, None]

    out_ref[...] = jnp.broadcast_to(
        y3_ref[pl.ds(i * _BLOCK_C, _BLOCK_C)], out_ref.shape
    )


@jax.jit
def kernel(pos, sensor_type, emb, W1, b1, W2, b2, g):
    B, C = pos.shape[0], pos.shape[1]
    D = emb.shape[1]

    x_cdb = pl.pallas_call(
        _fused_kernel,
        grid=(C // _BLOCK_C,),
        in_specs=[
            pl.BlockSpec(emb.shape, lambda i: (0, 0)),
            pl.BlockSpec(W1.shape, lambda i: (0, 0)),
            pl.BlockSpec((1, b1.shape[0]), lambda i: (0, 0)),
            pl.BlockSpec(W2.shape, lambda i: (0, 0)),
            pl.BlockSpec((1, b2.shape[0]), lambda i: (0, 0)),
            pl.BlockSpec((1, g.shape[0]), lambda i: (0, 0)),
        ],
        out_specs=pl.BlockSpec((_BLOCK_C, D, B), lambda i: (i, 0, 0)),
        out_shape=jax.ShapeDtypeStruct((C, D, B), jnp.float32),
        scratch_shapes=[pltpu.VMEM((C, D, 1), jnp.float32)],
    )(emb, W1, b1.reshape(1, -1), W2, b2.reshape(1, -1), g.reshape(1, -1))

    return jnp.transpose(x_cdb, (2, 0, 1))


# speedup vs baseline: 1.0539x; 1.0172x over previous
import jax
import jax.numpy as jnp
from jax.experimental import pallas as pl
from jax.experimental.pallas import tpu as pltpu

_BLOCK_C = 6


def _fused_kernel(embT_ref, W1T_ref, b1_ref, W2T_ref, b2_ref, g_ref, out_ref, y3_ref):
    C = y3_ref.shape[0]
    i = pl.program_id(0)

    @pl.when(i == 0)
    def _compute_tile():
        x = embT_ref[:, 0:C].T
        W1 = W1T_ref[...].T
        W2 = W2T_ref[...].T
        h = jnp.dot(x, W1, preferred_element_type=jnp.float32) + b1_ref[...]
        h = jax.nn.gelu(h)
        h = jnp.dot(h, W2, preferred_element_type=jnp.float32) + b2_ref[...]
        x = x + h
        ms = jnp.mean(x * x, axis=-1, keepdims=True)
        y = x * jax.lax.rsqrt(ms + 1e-6) * g_ref[...]
        y3_ref[...] = y[:, :, None]

    out_ref[...] = jnp.broadcast_to(
        y3_ref[pl.ds(i * _BLOCK_C, _BLOCK_C)], out_ref.shape
    )


@jax.jit
def kernel(pos, sensor_type, emb, W1, b1, W2, b2, g):
    B, C = pos.shape[0], pos.shape[1]
    D = emb.shape[1]

    x_cdb = pl.pallas_call(
        _fused_kernel,
        grid=(C // _BLOCK_C,),
        in_specs=[
            pl.BlockSpec((D, emb.shape[0]), lambda i: (0, 0)),
            pl.BlockSpec((W1.shape[1], W1.shape[0]), lambda i: (0, 0)),
            pl.BlockSpec((1, b1.shape[0]), lambda i: (0, 0)),
            pl.BlockSpec((W2.shape[1], W2.shape[0]), lambda i: (0, 0)),
            pl.BlockSpec((1, b2.shape[0]), lambda i: (0, 0)),
            pl.BlockSpec((1, g.shape[0]), lambda i: (0, 0)),
        ],
        out_specs=pl.BlockSpec((_BLOCK_C, D, B), lambda i: (i, 0, 0)),
        out_shape=jax.ShapeDtypeStruct((C, D, B), jnp.float32),
        scratch_shapes=[pltpu.VMEM((C, D, 1), jnp.float32)],
    )(emb.T, W1.T, b1.reshape(1, -1), W2.T, b2.reshape(1, -1), g.reshape(1, -1))

    return jnp.transpose(x_cdb, (2, 0, 1))
